# baseline (speedup 1.0000x reference)
import jax
import jax.numpy as jnp
from jax import lax
from jax.experimental import pallas as pl
from jax.experimental.pallas import tpu as pltpu
from jax.experimental.pallas import tpu_sc as plsc

_NT = 16
_NC = 2
_K = 128
_GA = 0.044715
_B2N = -1.5957691216057308


def _node_body(a_ref, b_ref, Ws_ref, bs_ref, Wd_ref, bd_ref, oxs_ref, oxd_ref):
    oxs_ref[...] = jnp.dot(a_ref[...], Ws_ref[...],
                           preferred_element_type=jnp.float32) + bs_ref[...]
    oxd_ref[...] = jnp.dot(b_ref[...], Wd_ref[...],
                           preferred_element_type=jnp.float32) + bd_ref[...]


def _edge_body(ea_ref, W1_ref, b1_ref, W2_ref, b2_ref, oe_ref):
    h = jnp.maximum(jnp.dot(ea_ref[...], W1_ref[...],
                            preferred_element_type=jnp.float32) + b1_ref[...], 0.0)
    oe_ref[...] = jnp.dot(h, W2_ref[...],
                          preferred_element_type=jnp.float32) + b2_ref[...]


def _mlp_body(pk_ref, xd_ref, W1_ref, b1_ref, W2_ref, b2_ref, out_ref):
    a = pk_ref[0] + pk_ref[1] + xd_ref[...]
    h = jnp.maximum(jnp.dot(a, W1_ref[...],
                            preferred_element_type=jnp.float32) + b1_ref[...], 0.0)
    out_ref[...] = jnp.dot(h, W2_ref[...],
                           preferred_element_type=jnp.float32) + b2_ref[...]


def _sc_body(xs_hbm, e_hbm, src_hbm, dst_hbm, w_hbm, out_hbm,
             agg_sh, srcbA, dstbA, srcbB, dstbB, wb, eb, gb,
             sem_idx, sem_ew, sem_g):
    c = lax.axis_index("c")
    s = lax.axis_index("s")
    n = agg_sh.shape[0]
    rpt = n // _NT
    r0 = s * rpt
    epc = (e_hbm.shape[0]) // _NC
    ept = epc // _NT
    nch = ept // _K
    slab0 = c * (epc // _K) + s * nch

    def zrow(r, carry):
        z = jnp.zeros((16,), jnp.float32)
        for ccol in range(8):
            gb[r, pl.ds(ccol * 16, 16)] = z
        return carry
    lax.fori_loop(0, _K, zrow, 0)
    nz = rpt // _K
    for zi in range(nz):
        pltpu.sync_copy(gb, agg_sh.at[pl.ds(r0 + zi * _K, _K)])
    plsc.subcore_barrier()

    bufs = ((srcbA, dstbA), (srcbB, dstbB))
    for j, (sb, db) in enumerate(bufs):
        pltpu.async_copy(src_hbm.at[slab0 + j], sb, sem_idx)
        pltpu.async_copy(dst_hbm.at[slab0 + j], db, sem_idx)

    def pair(i2, carry):
        for j, (sb, db) in enumerate(bufs):
            i = 2 * i2 + j
            base = c * epc + s * ept + i * _K
            slab = slab0 + i
            wrow = c * (epc // 8) + s * (ept // 8) + i * (_K // 8)
            pltpu.make_async_copy(src_hbm.at[slab], sb, sem_idx).wait()
            pltpu.make_async_copy(dst_hbm.at[slab], db, sem_idx).wait()
            g = pltpu.async_copy(xs_hbm.at[sb.at[0]], gb, sem_g)
            ce = pltpu.async_copy(e_hbm.at[pl.ds(base, _K)], eb, sem_ew)
            cw = pltpu.async_copy(w_hbm.at[pl.ds(wrow, _K // 8)], wb, sem_ew)
            g.wait()
            pltpu.async_copy(src_hbm.at[slab + 2], sb, sem_idx)
            ce.wait()
            cw.wait()

            def grpf(g8, cc):
                for j8 in range(8):
                    r = g8 * 8 + j8
                    wv = wb[g8, pl.ds(j8 * 16, 16)]
                    for ccol in range(8):
                        sl = pl.ds(ccol * 16, 16)
                        x = gb[r, sl] + eb[r, sl]
                        x3 = x * x * x
                        q = jnp.exp(_B2N * (x + _GA * x3))
                        gb[r, sl] = (x * wv) / (1.0 + q)
                return cc
            lax.fori_loop(0, _K // 8, grpf, 0)

            pltpu.sync_copy(gb, agg_sh.at[db.at[0]], add=True)
            pltpu.async_copy(dst_hbm.at[slab + 2], db, sem_idx)
        return carry
    lax.fori_loop(0, nch // 2, pair, 0)

    for j, (sb, db) in enumerate(bufs):
        pltpu.make_async_copy(src_hbm.at[slab0 + j], sb, sem_idx).wait()
        pltpu.make_async_copy(dst_hbm.at[slab0 + j], db, sem_idx).wait()

    plsc.subcore_barrier()
    pltpu.sync_copy(agg_sh.at[pl.ds(r0, rpt)], out_hbm.at[c, pl.ds(r0, rpt)])


def kernel(x_src, x_dst, edge_index, edge_attr, edge_weight,
           W_src, b_src, W_dst, b_dst, We1, be1, We2, be2,
           Wm1, bm1, Wm2, bm2, eps):
    N, D = x_src.shape
    E = edge_attr.shape[0]
    DE = edge_attr.shape[1]
    H = W_src.shape[1]
    f32 = jnp.float32

    src = edge_index[0].astype(jnp.int32)
    dst = edge_index[1].astype(jnp.int32)
    W_dst_eff = (1.0 + eps) * W_dst
    grp = _NC * _NT * _K * 2
    E_pad = ((E + grp - 1) // grp) * grp
    pe = E_pad - E
    src_p = jnp.pad(src, (0, pe + 2 * _K)).reshape(E_pad // _K + 2, 1, _K)
    dst_p = jnp.pad(dst, (0, pe + 2 * _K)).reshape(E_pad // _K + 2, 1, _K)
    ew_p = jnp.pad(edge_weight, (0, pe))
    ew16 = jnp.broadcast_to(ew_p[:, None], (E_pad, 16)).reshape(E_pad // 8, 128)
    ea_p = jnp.pad(edge_attr, ((0, pe), (0, 0)))
    N_pad = ((N + _NT * 128 - 1) // (_NT * 128)) * (_NT * 128)
    pn = N_pad - N
    xsrc_p = jnp.pad(x_src, ((0, pn), (0, 0)))
    xdst_p = jnp.pad(x_dst, ((0, pn), (0, 0)))

    BN = 2048
    BE = 2048

    xs, xd = pl.pallas_call(
        _node_body,
        grid=(N_pad // BN,),
        in_specs=[
            pl.BlockSpec((BN, D), lambda i: (i, 0)),
            pl.BlockSpec((BN, D), lambda i: (i, 0)),
            pl.BlockSpec((D, H), lambda i: (0, 0)),
            pl.BlockSpec((1, H), lambda i: (0, 0)),
            pl.BlockSpec((D, H), lambda i: (0, 0)),
            pl.BlockSpec((1, H), lambda i: (0, 0)),
        ],
        out_specs=[
            pl.BlockSpec((BN, H), lambda i: (i, 0)),
            pl.BlockSpec((BN, H), lambda i: (i, 0)),
        ],
        out_shape=[
            jax.ShapeDtypeStruct((N_pad, H), f32),
            jax.ShapeDtypeStruct((N_pad, H), f32),
        ],
    )(xsrc_p, xdst_p, W_src, b_src.reshape(1, H), W_dst_eff, b_dst.reshape(1, H))

    e_p = pl.pallas_call(
        _edge_body,
        grid=(E_pad // BE,),
        in_specs=[
            pl.BlockSpec((BE, DE), lambda i: (i, 0)),
            pl.BlockSpec((DE, H), lambda i: (0, 0)),
            pl.BlockSpec((1, H), lambda i: (0, 0)),
            pl.BlockSpec((H, H), lambda i: (0, 0)),
            pl.BlockSpec((1, H), lambda i: (0, 0)),
        ],
        out_specs=pl.BlockSpec((BE, H), lambda i: (i, 0)),
        out_shape=jax.ShapeDtypeStruct((E_pad, H), f32),
    )(ea_p, We1, be1.reshape(1, H), We2, be2.reshape(1, H))

    mesh = plsc.VectorSubcoreMesh(core_axis_name="c", subcore_axis_name="s",
                                  num_cores=_NC, num_subcores=_NT)
    sc_fn = pl.kernel(
        _sc_body,
        out_type=jax.ShapeDtypeStruct((_NC, N_pad, H), f32),
        mesh=mesh,
        scratch_types=[
            pltpu.VMEM_SHARED((N_pad, H), f32),
            pltpu.VMEM((1, _K), jnp.int32),
            pltpu.VMEM((1, _K), jnp.int32),
            pltpu.VMEM((1, _K), jnp.int32),
            pltpu.VMEM((1, _K), jnp.int32),
            pltpu.VMEM((_K // 8, 128), f32),
            pltpu.VMEM((_K, H), f32),
            pltpu.VMEM((_K, H), f32),
            pltpu.SemaphoreType.DMA,
            pltpu.SemaphoreType.DMA,
            pltpu.SemaphoreType.DMA,
        ],
    )
    aggr_pk = sc_fn(xs, e_p, src_p, dst_p, ew16)

    out_p = pl.pallas_call(
        _mlp_body,
        grid=(N_pad // BN,),
        in_specs=[
            pl.BlockSpec((_NC, BN, H), lambda i: (0, i, 0)),
            pl.BlockSpec((BN, H), lambda i: (i, 0)),
            pl.BlockSpec((H, H), lambda i: (0, 0)),
            pl.BlockSpec((1, H), lambda i: (0, 0)),
            pl.BlockSpec((H, H), lambda i: (0, 0)),
            pl.BlockSpec((1, H), lambda i: (0, 0)),
        ],
        out_specs=pl.BlockSpec((BN, H), lambda i: (i, 0)),
        out_shape=jax.ShapeDtypeStruct((N_pad, H), f32),
    )(aggr_pk, xd, Wm1, bm1.reshape(1, H), Wm2, bm2.reshape(1, H))
    return out_p[:N]


# speedup vs baseline: 1.2841x; 1.2841x over previous
import jax
import jax.numpy as jnp
from jax import lax
from jax.experimental import pallas as pl
from jax.experimental.pallas import tpu as pltpu
from jax.experimental.pallas import tpu_sc as plsc

_NT = 16
_NC = 2
_K = 128
_GA = 0.044715
_B2N = -1.5957691216057308


def _node_body(a_ref, b_ref, Ws_ref, bs_ref, Wd_ref, bd_ref, oxs_ref, oxd_ref):
    oxs_ref[...] = jnp.dot(a_ref[...], Ws_ref[...],
                           preferred_element_type=jnp.float32) + bs_ref[...]
    oxd_ref[...] = jnp.dot(b_ref[...], Wd_ref[...],
                           preferred_element_type=jnp.float32) + bd_ref[...]


def _edge_body(ea_ref, W1_ref, b1_ref, W2_ref, b2_ref, oe_ref):
    h = jnp.maximum(jnp.dot(ea_ref[...], W1_ref[...],
                            preferred_element_type=jnp.float32) + b1_ref[...], 0.0)
    oe_ref[...] = jnp.dot(h, W2_ref[...],
                          preferred_element_type=jnp.float32) + b2_ref[...]


def _mlp_body(pk_ref, xd_ref, W1_ref, b1_ref, W2_ref, b2_ref, out_ref):
    a = pk_ref[0] + pk_ref[1] + xd_ref[...]
    h = jnp.maximum(jnp.dot(a, W1_ref[...],
                            preferred_element_type=jnp.float32) + b1_ref[...], 0.0)
    out_ref[...] = jnp.dot(h, W2_ref[...],
                           preferred_element_type=jnp.float32) + b2_ref[...]


def _sc_body(xs_hbm, e_hbm, src_hbm, dst_hbm, w_hbm, out_hbm,
             agg_sh, srcbA, dstbA, srcbB, dstbB, wb, eb, gb,
             sem_idx, sem_ew, sem_g):
    c = lax.axis_index("c")
    s = lax.axis_index("s")
    n = agg_sh.shape[0]
    rpt = n // _NT
    r0 = s * rpt
    epc = (e_hbm.shape[0]) // _NC
    ept = epc // _NT
    nch = ept // _K
    slab0 = c * (epc // _K) + s * nch

    def zrow(r, carry):
        z = jnp.zeros((16,), jnp.float32)
        for ccol in range(8):
            gb[r, pl.ds(ccol * 16, 16)] = z
        return carry
    lax.fori_loop(0, _K, zrow, 0)
    nz = rpt // _K
    for zi in range(nz):
        pltpu.sync_copy(gb, agg_sh.at[pl.ds(r0 + zi * _K, _K)])
    plsc.subcore_barrier()

    def chunk(i, carry):
        base = c * epc + s * ept + i * _K
        slab = slab0 + i
        wrow = c * (epc // 8) + s * (ept // 8) + i * (_K // 8)
        with jax.named_scope("ph_lin"):
            cps = [
                pltpu.async_copy(src_hbm.at[slab], srcbA, sem_idx),
                pltpu.async_copy(dst_hbm.at[slab], dstbA, sem_idx),
                pltpu.async_copy(w_hbm.at[pl.ds(wrow, _K // 8)], wb, sem_ew),
                pltpu.async_copy(e_hbm.at[pl.ds(base, _K)], eb, sem_ew),
            ]
            for cp in cps:
                cp.wait()
        with jax.named_scope("ph_gth"):
            pltpu.async_copy(xs_hbm.at[srcbA.at[0]], gb, sem_g).wait()

        with jax.named_scope("ph_cmp"):
            def grpf(g8, cc):
                for j8 in range(8):
                    r = g8 * 8 + j8
                    wv = wb[g8, pl.ds(j8 * 16, 16)]
                    for ccol in range(8):
                        sl = pl.ds(ccol * 16, 16)
                        x = gb[r, sl] + eb[r, sl]
                        x3 = x * x * x
                        q = jnp.exp(_B2N * (x + _GA * x3))
                        gb[r, sl] = (x * wv) / (1.0 + q)
                return cc
            lax.fori_loop(0, _K // 8, grpf, 0)

        with jax.named_scope("ph_sct"):
            pltpu.sync_copy(gb, agg_sh.at[dstbA.at[0]], add=True)
        return carry
    lax.fori_loop(0, nch, chunk, 0)

    plsc.subcore_barrier()
    pltpu.sync_copy(agg_sh.at[pl.ds(r0, rpt)], out_hbm.at[c, pl.ds(r0, rpt)])


def kernel(x_src, x_dst, edge_index, edge_attr, edge_weight,
           W_src, b_src, W_dst, b_dst, We1, be1, We2, be2,
           Wm1, bm1, Wm2, bm2, eps):
    N, D = x_src.shape
    E = edge_attr.shape[0]
    DE = edge_attr.shape[1]
    H = W_src.shape[1]
    f32 = jnp.float32

    src = edge_index[0].astype(jnp.int32)
    dst = edge_index[1].astype(jnp.int32)
    W_dst_eff = (1.0 + eps) * W_dst
    grp = _NC * _NT * _K * 2
    E_pad = ((E + grp - 1) // grp) * grp
    pe = E_pad - E
    src_p = jnp.pad(src, (0, pe + 2 * _K)).reshape(E_pad // _K + 2, 1, _K)
    dst_p = jnp.pad(dst, (0, pe + 2 * _K)).reshape(E_pad // _K + 2, 1, _K)
    ew_p = jnp.pad(edge_weight, (0, pe))
    ew16 = jnp.broadcast_to(ew_p[:, None], (E_pad, 16)).reshape(E_pad // 8, 128)
    ea_p = jnp.pad(edge_attr, ((0, pe), (0, 0)))
    N_pad = ((N + _NT * 128 - 1) // (_NT * 128)) * (_NT * 128)
    pn = N_pad - N
    xsrc_p = jnp.pad(x_src, ((0, pn), (0, 0)))
    xdst_p = jnp.pad(x_dst, ((0, pn), (0, 0)))

    BN = 2048
    BE = 2048

    xs, xd = pl.pallas_call(
        _node_body,
        grid=(N_pad // BN,),
        in_specs=[
            pl.BlockSpec((BN, D), lambda i: (i, 0)),
            pl.BlockSpec((BN, D), lambda i: (i, 0)),
            pl.BlockSpec((D, H), lambda i: (0, 0)),
            pl.BlockSpec((1, H), lambda i: (0, 0)),
            pl.BlockSpec((D, H), lambda i: (0, 0)),
            pl.BlockSpec((1, H), lambda i: (0, 0)),
        ],
        out_specs=[
            pl.BlockSpec((BN, H), lambda i: (i, 0)),
            pl.BlockSpec((BN, H), lambda i: (i, 0)),
        ],
        out_shape=[
            jax.ShapeDtypeStruct((N_pad, H), f32),
            jax.ShapeDtypeStruct((N_pad, H), f32),
        ],
    )(xsrc_p, xdst_p, W_src, b_src.reshape(1, H), W_dst_eff, b_dst.reshape(1, H))

    e_p = pl.pallas_call(
        _edge_body,
        grid=(E_pad // BE,),
        in_specs=[
            pl.BlockSpec((BE, DE), lambda i: (i, 0)),
            pl.BlockSpec((DE, H), lambda i: (0, 0)),
            pl.BlockSpec((1, H), lambda i: (0, 0)),
            pl.BlockSpec((H, H), lambda i: (0, 0)),
            pl.BlockSpec((1, H), lambda i: (0, 0)),
        ],
        out_specs=pl.BlockSpec((BE, H), lambda i: (i, 0)),
        out_shape=jax.ShapeDtypeStruct((E_pad, H), f32),
    )(ea_p, We1, be1.reshape(1, H), We2, be2.reshape(1, H))

    mesh = plsc.VectorSubcoreMesh(core_axis_name="c", subcore_axis_name="s",
                                  num_cores=_NC, num_subcores=_NT)
    sc_fn = pl.kernel(
        _sc_body,
        out_type=jax.ShapeDtypeStruct((_NC, N_pad, H), f32),
        mesh=mesh,
        scratch_types=[
            pltpu.VMEM_SHARED((N_pad, H), f32),
            pltpu.VMEM((1, _K), jnp.int32),
            pltpu.VMEM((1, _K), jnp.int32),
            pltpu.VMEM((1, _K), jnp.int32),
            pltpu.VMEM((1, _K), jnp.int32),
            pltpu.VMEM((_K // 8, 128), f32),
            pltpu.VMEM((_K, H), f32),
            pltpu.VMEM((_K, H), f32),
            pltpu.SemaphoreType.DMA,
            pltpu.SemaphoreType.DMA,
            pltpu.SemaphoreType.DMA,
        ],
    )
    aggr_pk = sc_fn(xs, e_p, src_p, dst_p, ew16)

    out_p = pl.pallas_call(
        _mlp_body,
        grid=(N_pad // BN,),
        in_specs=[
            pl.BlockSpec((_NC, BN, H), lambda i: (0, i, 0)),
            pl.BlockSpec((BN, H), lambda i: (i, 0)),
            pl.BlockSpec((H, H), lambda i: (0, 0)),
            pl.BlockSpec((1, H), lambda i: (0, 0)),
            pl.BlockSpec((H, H), lambda i: (0, 0)),
            pl.BlockSpec((1, H), lambda i: (0, 0)),
        ],
        out_specs=pl.BlockSpec((BN, H), lambda i: (i, 0)),
        out_shape=jax.ShapeDtypeStruct((N_pad, H), f32),
    )(aggr_pk, xd, Wm1, bm1.reshape(1, H), Wm2, bm2.reshape(1, H))
    return out_p[:N]
